# baseline (speedup 1.0000x reference)
import jax
import jax.numpy as jnp
from jax.experimental import pallas as pl
from jax.experimental.pallas import tpu as pltpu

N = 32768
S = 2048
K = 64
RADIUS = 0.2

BR = 256
BC = 2048


def _dist_block(cent_ref, vt_ref, sq_ref, d_ref):
    cp = cent_ref[...]
    vt = vt_ref[...]
    sq = sq_ref[...][0:1, :]
    csq = cp[:, 3:4]
    t = jax.lax.dot_general(cp, vt, (((1,), (0,)), ((), ())))
    d_ref[...] = jnp.sqrt(jnp.abs(csq - 2.0 * t + sq))


def _distances(cent_pad, vt, sqr):
    grid = (S // BR, N // BC)
    return pl.pallas_call(
        _dist_block,
        grid=grid,
        in_specs=[
            pl.BlockSpec((BR, 128), lambda i, j: (i, 0)),
            pl.BlockSpec((128, BC), lambda i, j: (0, j)),
            pl.BlockSpec((8, BC), lambda i, j: (0, j)),
        ],
        out_specs=pl.BlockSpec((BR, BC), lambda i, j: (i, j)),
        out_shape=jax.ShapeDtypeStruct((S, N), jnp.float32),
    )(cent_pad, vt, sqr)


def _fps_body(vx_ref, vy_ref, vz_ref, cent_ref, md_ref):
    vx = vx_ref[...]
    vy = vy_ref[...]
    vz = vz_ref[...]
    rows = jax.lax.broadcasted_iota(jnp.int32, (N // 128, 128), 0)
    cols = jax.lax.broadcasted_iota(jnp.int32, (N // 128, 128), 1)
    idx2 = rows * 128 + cols
    md_ref[...] = jnp.full((N // 128, 128), jnp.inf, dtype=jnp.float32)

    def select(nxt):
        m = idx2 == nxt
        lx = jnp.sum(jnp.where(m, vx, 0.0))
        ly = jnp.sum(jnp.where(m, vy, 0.0))
        lz = jnp.sum(jnp.where(m, vz, 0.0))
        return lx, ly, lz

    def write_row(i, nxt, lx, ly, lz):
        csq = (lx * lx + ly * ly) + lz * lz
        lane = jax.lax.broadcasted_iota(jnp.int32, (1, 128), 1)
        row = jnp.where(lane == 0, lx, 0.0)
        row = jnp.where(lane == 1, ly, row)
        row = jnp.where(lane == 2, lz, row)
        row = jnp.where(lane == 3, csq, row)
        row = jnp.where(lane == 4, nxt.astype(jnp.float32), row)
        cent_ref[pl.ds(i, 1), :] = row

    lx0, ly0, lz0 = select(jnp.int32(0))
    write_row(0, jnp.int32(0), lx0, ly0, lz0)

    def body(i, carry):
        lx, ly, lz = carry
        dx = vx - lx
        dy = vy - ly
        dz = vz - lz
        d = (dx * dx + dy * dy) + dz * dz
        nmd = jnp.minimum(md_ref[...], d)
        md_ref[...] = nmd
        mx = jnp.max(nmd)
        nxt = jnp.min(jnp.where(nmd == mx, idx2, N))
        nlx, nly, nlz = select(nxt)
        write_row(i, nxt, nlx, nly, nlz)
        return nlx, nly, nlz

    jax.lax.fori_loop(1, S, body, (lx0, ly0, lz0))


def _fps_pallas(vx, vy, vz):
    return pl.pallas_call(
        _fps_body,
        out_shape=jax.ShapeDtypeStruct((S, 128), jnp.float32),
        scratch_shapes=[pltpu.VMEM((N // 128, 128), jnp.float32)],
    )(vx, vy, vz)


def _fps_xla(vertices, n_samples):
    n = vertices.shape[0]
    idxs = jnp.zeros((n_samples,), dtype=jnp.int32)
    min_d = jnp.full((n,), jnp.inf, dtype=vertices.dtype)

    def body(i, state):
        idxs, min_d = state
        last = vertices[idxs[i - 1]]
        d = jnp.sum((vertices - last) ** 2, axis=-1)
        min_d = jnp.minimum(min_d, d)
        nxt = jnp.argmax(min_d).astype(jnp.int32)
        return idxs.at[i].set(nxt), min_d

    idxs, _ = jax.lax.fori_loop(1, n_samples, body, (idxs, min_d))
    return idxs


def kernel(vertex_features, vertices):
    del vertex_features
    vtr = vertices.T
    vx = vtr[0].reshape(N // 128, 128)
    vy = vtr[1].reshape(N // 128, 128)
    vz = vtr[2].reshape(N // 128, 128)
    cent_pad = _fps_pallas(vx, vy, vz)
    centroid_idx = cent_pad[:, 4].astype(jnp.int32)
    sq = jnp.einsum('ij,ij->i', vertices, vertices)
    vt = jnp.zeros((128, N), jnp.float32).at[0:3, :].set(vtr)
    sqr = jnp.zeros((8, N), jnp.float32).at[0, :].set(sq)
    d = _distances(cent_pad, vt, sqr)
    neg_d, nbr_idx = jax.lax.top_k(-d, K)
    limits = jnp.minimum(-neg_d[:, K - 1], RADIUS)
    valid = (-neg_d) <= limits[:, None]
    nbr_idx = jnp.where(valid, nbr_idx, centroid_idx[:, None])
    return jnp.take(vertices, nbr_idx, axis=0)
